# baseline (speedup 1.0000x reference)
import jax
import jax.numpy as jnp
from jax.experimental import pallas as pl
from jax.experimental.pallas import tpu as pltpu

_BM = 256


def _pathcon_body(x_ref, w_ref, b_ref, scores_ref, sig_ref, acc_ref):
    i = pl.program_id(0)
    n = pl.num_programs(0)

    @pl.when(i > 0)
    def _epilogue():
        scores = acc_ref[(i - 1) % 2] + b_ref[...]
        scores_ref[...] = scores
        sig_ref[...] = jax.nn.sigmoid(scores)

    @pl.when(i < n - 1)
    def _matmul():
        acc_ref[i % 2] = jax.lax.dot_general(
            w_ref[...], x_ref[...],
            dimension_numbers=(((1,), (1,)), ((), ())),
            preferred_element_type=jnp.float32,
        )


def kernel(path_features, labels, W, b):
    del labels
    batch, n_paths = path_features.shape
    n_rel = W.shape[0]
    b2 = b.reshape(n_rel, 1)

    nblk = batch // _BM
    grid = (nblk + 1,)
    out_shape = [
        jax.ShapeDtypeStruct((n_rel, batch), jnp.float32),
        jax.ShapeDtypeStruct((n_rel, batch), jnp.float32),
    ]
    scores_t, sig_t = pl.pallas_call(
        _pathcon_body,
        grid=grid,
        in_specs=[
            pl.BlockSpec((_BM, n_paths), lambda i: (jnp.minimum(i, nblk - 1), 0)),
            pl.BlockSpec((n_rel, n_paths), lambda i: (0, 0)),
            pl.BlockSpec((n_rel, 1), lambda i: (0, 0)),
        ],
        out_specs=[
            pl.BlockSpec((n_rel, _BM), lambda i: (0, jnp.maximum(i - 1, 0))),
            pl.BlockSpec((n_rel, _BM), lambda i: (0, jnp.maximum(i - 1, 0))),
        ],
        out_shape=out_shape,
        scratch_shapes=[pltpu.VMEM((2, n_rel, _BM), jnp.float32)],
        compiler_params=pltpu.CompilerParams(
            dimension_semantics=("arbitrary",),
        ),
    )(path_features, W, b2)
    return (jnp.swapaxes(scores_t, 0, 1), jnp.swapaxes(sig_t, 0, 1))
